# Initial kernel scaffold; baseline (speedup 1.0000x reference)
import jax
import jax.numpy as jnp
from jax.experimental import pallas as pl


def kernel(ui_edge_index, ub_edge_index, bi_edge_index, users_feature, items_feature, bundles_feature, W1_item, W2_item, W1_bundle, W2_bundle, gate_W, gate_b):
    raise NotImplementedError("write your pallas kernel here")


# speedup vs baseline: 2.0288x; 2.0288x over previous
import jax
import jax.numpy as jnp
from jax.experimental import pallas as pl
from jax.experimental.pallas import tpu as pltpu

_U, _I, _B, _D = 10000, 10000, 10000, 256
_E = 160000
_EPS = 1e-8


def _scatter_rows(hp, src, dst, n_out):
    return jax.ops.segment_sum(hp[src], dst, num_segments=n_out)


def _count(dst, n_out):
    return jax.ops.segment_sum(jnp.ones(dst.shape[0], jnp.float32), dst,
                               num_segments=n_out)


def _prep_tc(deg, x):
    N = x.shape[0]
    BLK = 1000

    def body(deg_ref, x_ref, a_ref, hp_ref):
        a = 1.0 / (jnp.sqrt(deg_ref[...]) + _EPS)
        a_ref[...] = a
        hp_ref[...] = a * x_ref[...]

    return pl.pallas_call(
        body,
        grid=(N // BLK,),
        in_specs=[pl.BlockSpec((BLK, 1), lambda i: (i, 0)),
                  pl.BlockSpec((BLK, _D), lambda i: (i, 0))],
        out_specs=[pl.BlockSpec((BLK, 1), lambda i: (i, 0)),
                   pl.BlockSpec((BLK, _D), lambda i: (i, 0))],
        out_shape=[jax.ShapeDtypeStruct((N, 1), jnp.float32),
                   jax.ShapeDtypeStruct((N, _D), jnp.float32)],
    )(deg, x)


def _layer1_tc(s, a, h, W):
    N = s.shape[0]
    BLK = 1000

    def body(s_ref, a_ref, h_ref, w_ref, h1_ref, hp1_ref):
        aa = a_ref[...]
        m = aa * (s_ref[...] + aa * h_ref[...])
        h1 = jnp.tanh(jnp.dot(m, w_ref[...], preferred_element_type=jnp.float32))
        h1_ref[...] = h1
        hp1_ref[...] = aa * h1

    return pl.pallas_call(
        body,
        grid=(N // BLK,),
        in_specs=[pl.BlockSpec((BLK, _D), lambda i: (i, 0)),
                  pl.BlockSpec((BLK, 1), lambda i: (i, 0)),
                  pl.BlockSpec((BLK, _D), lambda i: (i, 0)),
                  pl.BlockSpec((_D, _D), lambda i: (0, 0))],
        out_specs=[pl.BlockSpec((BLK, _D), lambda i: (i, 0)),
                   pl.BlockSpec((BLK, _D), lambda i: (i, 0))],
        out_shape=[jax.ShapeDtypeStruct((N, _D), jnp.float32),
                   jax.ShapeDtypeStruct((N, _D), jnp.float32)],
    )(s, a, h, W)


def _layer2_tc(s, a, h1, x, W):
    N = s.shape[0]
    BLK = 1000

    def body(s_ref, a_ref, h1_ref, x_ref, w_ref, o_ref):
        aa = a_ref[...]
        h1 = h1_ref[...]
        m = aa * (s_ref[...] + aa * h1)
        h2 = jnp.tanh(jnp.dot(m, w_ref[...], preferred_element_type=jnp.float32))
        o_ref[...] = (x_ref[...] + h1 + h2) * (1.0 / 3.0)

    return pl.pallas_call(
        body,
        grid=(N // BLK,),
        in_specs=[pl.BlockSpec((BLK, _D), lambda i: (i, 0)),
                  pl.BlockSpec((BLK, 1), lambda i: (i, 0)),
                  pl.BlockSpec((BLK, _D), lambda i: (i, 0)),
                  pl.BlockSpec((BLK, _D), lambda i: (i, 0)),
                  pl.BlockSpec((_D, _D), lambda i: (0, 0))],
        out_specs=pl.BlockSpec((BLK, _D), lambda i: (i, 0)),
        out_shape=jax.ShapeDtypeStruct((N, _D), jnp.float32),
    )(s, a, h1, x, W)


def _gate_tc(il, cnt, bl, feat, gate_W, gate_b):
    N = il.shape[0]
    BLK = 1000
    has_cnt = cnt is not None

    def body(*refs):
        if has_cnt:
            il_ref, cnt_ref, bl_ref, f_ref, w_ref, b_ref, o_ref = refs
            ilv = il_ref[...] / (cnt_ref[...] + _EPS)
        else:
            il_ref, bl_ref, f_ref, w_ref, b_ref, o_ref = refs
            ilv = il_ref[...]
        blv = bl_ref[...]
        g = (jnp.dot(ilv, w_ref[0:_D, :], preferred_element_type=jnp.float32)
             + jnp.dot(blv, w_ref[_D:2 * _D, :], preferred_element_type=jnp.float32)
             + jnp.dot(f_ref[...], w_ref[2 * _D:3 * _D, :], preferred_element_type=jnp.float32)
             + b_ref[...])
        m = jnp.max(g, axis=1, keepdims=True)
        e = jnp.exp(g - m)
        w0 = e[:, 0:1] / (e[:, 0:1] + e[:, 1:2])
        o_ref[...] = w0 * ilv + (1.0 - w0) * blv

    in_specs = [pl.BlockSpec((BLK, _D), lambda i: (i, 0))]
    args = [il]
    if has_cnt:
        in_specs.append(pl.BlockSpec((BLK, 1), lambda i: (i, 0)))
        args.append(cnt)
    in_specs += [pl.BlockSpec((BLK, _D), lambda i: (i, 0)),
                 pl.BlockSpec((BLK, _D), lambda i: (i, 0)),
                 pl.BlockSpec((3 * _D, 2), lambda i: (0, 0)),
                 pl.BlockSpec((1, 2), lambda i: (0, 0))]
    args += [bl, feat, gate_W, gate_b.reshape(1, 2)]

    return pl.pallas_call(
        body,
        grid=(N // BLK,),
        in_specs=in_specs,
        out_specs=pl.BlockSpec((BLK, _D), lambda i: (i, 0)),
        out_shape=jax.ShapeDtypeStruct((N, _D), jnp.float32),
    )(*args)


def _level(ei, featA, featB, W1, W2, NA, NB):
    N = NA + NB
    src = jnp.concatenate([ei[0], ei[1] + NA])
    dst = jnp.concatenate([ei[1] + NA, ei[0]])
    x = jnp.concatenate([featA, featB], axis=0)
    deg = (_count(dst, N) + 1.0).reshape(N, 1)
    a, hp0 = _prep_tc(deg, x)
    s1 = _scatter_rows(hp0, src, dst, N)
    h1, hp1 = _layer1_tc(s1, a, x, W1)
    s2 = _scatter_rows(hp1, src, dst, N)
    return _layer2_tc(s2, a, h1, x, W2)


def kernel(ui_edge_index, ub_edge_index, bi_edge_index, users_feature,
           items_feature, bundles_feature, W1_item, W2_item, W1_bundle,
           W2_bundle, gate_W, gate_b):
    out_ui = _level(ui_edge_index, users_feature, items_feature,
                    W1_item, W2_item, _U, _I)
    out_ub = _level(ub_edge_index, users_feature, bundles_feature,
                    W1_bundle, W2_bundle, _U, _B)
    users_il, items_il = out_ui[:_U], out_ui[_U:]
    users_bl, bundles_bl = out_ub[:_U], out_ub[_U:]

    b_idx, i_idx = bi_edge_index[0], bi_edge_index[1]
    pooled = _scatter_rows(items_il, i_idx, b_idx, _B)
    cnt = _count(b_idx, _B).reshape(_B, 1)

    users_out = _gate_tc(users_il, None, users_bl, users_feature,
                         gate_W, gate_b)
    bundles_out = _gate_tc(pooled, cnt, bundles_bl, bundles_feature,
                           gate_W, gate_b)
    return jnp.concatenate([users_out, bundles_out], axis=0)


# speedup vs baseline: 11.6557x; 5.7452x over previous
import functools

import jax
import jax.numpy as jnp
from jax import lax
from jax.experimental import pallas as pl
from jax.experimental.pallas import tpu as pltpu
from jax.experimental.pallas import tpu_sc as plsc

_U, _I, _B, _D = 10000, 10000, 10000, 256
_E = 160000
_EPS = 1e-8

_NR = 20480
_NRB = 10240
_BLK = 1024
_EB = 125

_MESH = plsc.VectorSubcoreMesh(core_axis_name="c", subcore_axis_name="s")


_HW = 16


def _hist_sc(dstb, npad):
    nblk = dstb.shape[0]
    bpt = nblk // 32
    grps = bpt // 4
    stripe = npad // 16

    @functools.partial(
        pl.kernel,
        out_type=jax.ShapeDtypeStruct((2 * npad, _HW), jnp.float32),
        mesh=_MESH,
        compiler_params=pltpu.CompilerParams(use_tc_tiling_on_sc=False),
        scratch_types=[
            pltpu.VMEM((bpt, _EB), jnp.int32),
            pltpu.VMEM((_EB, _HW), jnp.float32),
            pltpu.VMEM((128, _HW), jnp.float32),
            pltpu.VMEM_SHARED((npad, _HW), jnp.float32),
            pltpu.SemaphoreType.DMA,
        ],
    )
    def k(dstb_hbm, out_hbm, dstv, onesb, zbuf, acc, sem):
        cid = lax.axis_index("c")
        sid = lax.axis_index("s")
        wid = sid * 2 + cid
        ones16 = jnp.ones((16,), jnp.float32)
        zero16 = jnp.zeros((16,), jnp.float32)
        pltpu.sync_copy(dstb_hbm.at[pl.ds(wid * bpt, bpt)], dstv)

        def obody(i, _):
            onesb[i, pl.ds(0, 16)] = ones16
            return 0
        lax.fori_loop(0, _EB, obody, 0)

        def zbody(i, _):
            zbuf[i, pl.ds(0, 16)] = zero16
            return 0
        lax.fori_loop(0, 128, zbody, 0)

        for t in range(stripe // 128):
            pltpu.sync_copy(zbuf, acc.at[pl.ds(sid * stripe + t * 128, 128)])
        plsc.subcore_barrier()

        def body(n, _):
            for j in range(4):
                pltpu.async_copy(onesb, acc.at[dstv.at[n * 4 + j]], sem,
                                 add=True)
            for j in range(4):
                pltpu.make_async_copy(onesb, acc.at[dstv.at[0]], sem).wait()
            return 0
        lax.fori_loop(0, grps, body, 0)
        plsc.subcore_barrier()
        pltpu.sync_copy(acc.at[pl.ds(sid * stripe, stripe)],
                        out_hbm.at[pl.ds(cid * npad + sid * stripe, stripe)])

    return k(dstb)


def _scatter_sc(table, srcg, dstb, npad_out):
    nblk = dstb.shape[0]
    bpt = nblk // 16
    SG = 8
    sgrps = bpt // SG
    stripe = npad_out // 16
    zcop = stripe // 64

    @functools.partial(
        pl.kernel,
        out_type=jax.ShapeDtypeStruct((4 * npad_out, 64), jnp.float32),
        mesh=_MESH,
        compiler_params=pltpu.CompilerParams(use_tc_tiling_on_sc=False),
        scratch_types=[
            pltpu.VMEM((3, SG, _EB), jnp.int32),
            pltpu.VMEM((3, SG, _EB), jnp.int32),
            pltpu.VMEM((4, _EB, 64), jnp.float32),
            pltpu.VMEM((64, 64), jnp.float32),
            pltpu.VMEM_SHARED((npad_out, 64), jnp.float32),
            pltpu.SemaphoreType.DMA,
            pltpu.SemaphoreType.DMA,
            pltpu.SemaphoreType.DMA,
        ],
    )
    def k(tab_hbm, srcg_hbm, dstb_hbm, out_hbm,
          srcv, dstv, rowsb, zbuf, acc, sem_g, sem_s, sem_i):
        cid = lax.axis_index("c")
        sid = lax.axis_index("s")
        zero16 = jnp.zeros((16,), jnp.float32)
        tb0 = sid * bpt

        def zbody(i, _):
            r = lax.shift_right_logical(i, 2)
            c = lax.bitwise_and(i, 3)
            zbuf[r, pl.ds(c * 16, 16)] = zero16
            return 0
        lax.fori_loop(0, 256, zbody, 0)

        for kk in range(2):
            chunk = 2 * cid + kk

            def fire_is(s, par):
                pltpu.async_copy(
                    srcg_hbm.at[chunk, pl.ds(tb0 + s * SG, SG)],
                    srcv.at[par], sem_i)
                pltpu.async_copy(dstb_hbm.at[pl.ds(tb0 + s * SG, SG)],
                                 dstv.at[par], sem_i)

            def drain_is(par):
                for _ in range(2):
                    pltpu.make_async_copy(dstb_hbm.at[pl.ds(tb0, SG)],
                                          dstv.at[par], sem_i).wait()

            def fire_g(par, r):
                pltpu.async_copy(tab_hbm.at[srcv.at[par, r]],
                                 rowsb.at[r % 4], sem_g)

            def drain_g(r):
                pltpu.make_async_copy(tab_hbm.at[srcv.at[0, 0]],
                                      rowsb.at[r % 4], sem_g).wait()

            def fire_s(par, r):
                pltpu.async_copy(rowsb.at[r % 4], acc.at[dstv.at[par, r]],
                                 sem_s, add=True)

            def drain_s(r):
                pltpu.make_async_copy(rowsb.at[r % 4],
                                      acc.at[dstv.at[0, 0]], sem_s).wait()

            for t in range(zcop):
                pltpu.sync_copy(zbuf,
                                acc.at[pl.ds(sid * stripe + t * 64, 64)])
            plsc.subcore_barrier()

            def steady_rows(par, pp, first):
                for r in range(SG):
                    if not first or r >= 4:
                        drain_s(r % 4)
                    fire_g(par, r)
                    if first and r < 2:
                        continue
                    if r < 2:
                        drain_g((r - 2) % 4)
                        fire_s(pp, SG + r - 2)
                    else:
                        drain_g(r - 2)
                        fire_s(par, r - 2)

            fire_is(0, 0)
            drain_is(0)
            fire_is(1, 1)
            steady_rows(0, 0, True)

            def body(s, _):
                par = lax.rem(s, 3)
                pp = lax.rem(s + 2, 3)
                pn = lax.rem(s + 1, 3)
                drain_is(par)
                fire_is(s + 1, pn)
                steady_rows(par, pp, False)
                return 0
            lax.fori_loop(1, sgrps - 1, body, 0)

            pe = (sgrps - 1) % 3
            drain_is(pe)
            steady_rows(pe, (sgrps - 2) % 3, False)
            drain_g(2)
            fire_s(pe, SG - 2)
            drain_g(3)
            fire_s(pe, SG - 1)
            for r in range(4):
                drain_s(r)

            plsc.subcore_barrier()
            pltpu.sync_copy(
                acc.at[pl.ds(sid * stripe, stripe)],
                out_hbm.at[pl.ds(chunk * npad_out + sid * stripe, stripe)])
            plsc.subcore_barrier()

    return k(table, srcg, dstb)


def _prep_tc(hist, x):
    nr = x.shape[0]

    def body(d_ref, x_ref, a_ref, hp_ref):
        d = d_ref[0, :, 0:1] + d_ref[1, :, 0:1] + 1.0
        a = 1.0 / (jnp.sqrt(d) + _EPS)
        a_ref[...] = a
        hp = a * x_ref[...]
        for c in range(4):
            hp_ref[c] = hp[:, c * 64:(c + 1) * 64]

    return pl.pallas_call(
        body,
        grid=(nr // _BLK,),
        in_specs=[pl.BlockSpec((2, _BLK, _HW), lambda i: (0, i, 0)),
                  pl.BlockSpec((_BLK, _D), lambda i: (i, 0))],
        out_specs=[pl.BlockSpec((_BLK, 1), lambda i: (i, 0)),
                   pl.BlockSpec((4, _BLK, 64), lambda i: (0, i, 0))],
        out_shape=[jax.ShapeDtypeStruct((nr, 1), jnp.float32),
                   jax.ShapeDtypeStruct((4, nr, 64), jnp.float32)],
    )(hist, x)


def _layer1_tc(s4, a, h, W):
    nr = h.shape[0]

    def body(s_ref, a_ref, h_ref, w_ref, h1_ref, hp_ref):
        aa = a_ref[...]
        s = jnp.concatenate([s_ref[c] for c in range(4)], axis=1)
        m = aa * (s + aa * h_ref[...])
        h1 = jnp.tanh(jnp.dot(m, w_ref[...],
                              preferred_element_type=jnp.float32))
        h1_ref[...] = h1
        hp = aa * h1
        for c in range(4):
            hp_ref[c] = hp[:, c * 64:(c + 1) * 64]

    return pl.pallas_call(
        body,
        grid=(nr // _BLK,),
        in_specs=[pl.BlockSpec((4, _BLK, 64), lambda i: (0, i, 0)),
                  pl.BlockSpec((_BLK, 1), lambda i: (i, 0)),
                  pl.BlockSpec((_BLK, _D), lambda i: (i, 0)),
                  pl.BlockSpec((_D, _D), lambda i: (0, 0))],
        out_specs=[pl.BlockSpec((_BLK, _D), lambda i: (i, 0)),
                   pl.BlockSpec((4, _BLK, 64), lambda i: (0, i, 0))],
        out_shape=[jax.ShapeDtypeStruct((nr, _D), jnp.float32),
                   jax.ShapeDtypeStruct((4, nr, 64), jnp.float32)],
    )(s4, a, h, W)


def _layer2_tc(s4, a, h1, x, W, emit_chunked):
    nr = x.shape[0]

    def body(*refs):
        if emit_chunked:
            s_ref, a_ref, h1_ref, x_ref, w_ref, o_ref, oc_ref = refs
        else:
            s_ref, a_ref, h1_ref, x_ref, w_ref, o_ref = refs
        aa = a_ref[...]
        h1 = h1_ref[...]
        s = jnp.concatenate([s_ref[c] for c in range(4)], axis=1)
        m = aa * (s + aa * h1)
        h2 = jnp.tanh(jnp.dot(m, w_ref[...],
                              preferred_element_type=jnp.float32))
        o = (x_ref[...] + h1 + h2) * (1.0 / 3.0)
        o_ref[...] = o
        if emit_chunked:
            for c in range(4):
                oc_ref[c] = o[:, c * 64:(c + 1) * 64]

    out_specs = [pl.BlockSpec((_BLK, _D), lambda i: (i, 0))]
    out_shape = [jax.ShapeDtypeStruct((nr, _D), jnp.float32)]
    if emit_chunked:
        out_specs.append(pl.BlockSpec((4, _BLK, 64), lambda i: (0, i, 0)))
        out_shape.append(jax.ShapeDtypeStruct((4, nr, 64), jnp.float32))

    return pl.pallas_call(
        body,
        grid=(nr // _BLK,),
        in_specs=[pl.BlockSpec((4, _BLK, 64), lambda i: (0, i, 0)),
                  pl.BlockSpec((_BLK, 1), lambda i: (i, 0)),
                  pl.BlockSpec((_BLK, _D), lambda i: (i, 0)),
                  pl.BlockSpec((_BLK, _D), lambda i: (i, 0)),
                  pl.BlockSpec((_D, _D), lambda i: (0, 0))],
        out_specs=out_specs,
        out_shape=out_shape,
    )(s4, a, h1, x, W)


def _gate_tc(il, il4, cnt, bl, feat, gate_W, gate_b):
    nr = bl.shape[0]
    chunked = il4 is not None

    def body(*refs):
        if chunked:
            il_ref, cnt_ref, bl_ref, f_ref, w_ref, b_ref, o_ref = refs
            cntv = cnt_ref[0, :, 0:1] + cnt_ref[1, :, 0:1]
            ilv = jnp.concatenate([il_ref[c] for c in range(4)], axis=1)
            ilv = ilv / (cntv + _EPS)
        else:
            il_ref, bl_ref, f_ref, w_ref, b_ref, o_ref = refs
            ilv = il_ref[...]
        blv = bl_ref[...]
        g = (jnp.dot(ilv, w_ref[0:_D, :], preferred_element_type=jnp.float32)
             + jnp.dot(blv, w_ref[_D:2 * _D, :],
                       preferred_element_type=jnp.float32)
             + jnp.dot(f_ref[...], w_ref[2 * _D:3 * _D, :],
                       preferred_element_type=jnp.float32)
             + b_ref[...])
        m = jnp.max(g, axis=1, keepdims=True)
        e = jnp.exp(g - m)
        w0 = e[:, 0:1] / (e[:, 0:1] + e[:, 1:2])
        o_ref[...] = w0 * ilv + (1.0 - w0) * blv

    in_specs = []
    args = []
    if chunked:
        in_specs += [pl.BlockSpec((4, _BLK, 64), lambda i: (0, i, 0)),
                     pl.BlockSpec((2, _BLK, _HW), lambda i: (0, i, 0))]
        args += [il4, cnt]
    else:
        in_specs.append(pl.BlockSpec((_BLK, _D), lambda i: (i, 0)))
        args.append(il)
    in_specs += [pl.BlockSpec((_BLK, _D), lambda i: (i, 0)),
                 pl.BlockSpec((_BLK, _D), lambda i: (i, 0)),
                 pl.BlockSpec((3 * _D, 2), lambda i: (0, 0)),
                 pl.BlockSpec((1, 2), lambda i: (0, 0))]
    args += [bl, feat, gate_W, gate_b.reshape(1, 2)]

    return pl.pallas_call(
        body,
        grid=(nr // _BLK,),
        in_specs=in_specs,
        out_specs=pl.BlockSpec((_BLK, _D), lambda i: (i, 0)),
        out_shape=jax.ShapeDtypeStruct((nr, _D), jnp.float32),
    )(*args)


def _pad_rows(x, nr):
    return jnp.pad(x, ((0, nr - x.shape[0]), (0, 0)))


def _level(ei, featA, featB, W1, W2, emit_chunked):
    NA = featA.shape[0]
    src = jnp.concatenate([ei[0], ei[1] + NA])
    dst = jnp.concatenate([ei[1] + NA, ei[0]])
    offs = (jnp.arange(4, dtype=jnp.int32) * _NR)[:, None]
    srcg = (src[None, :] + offs).reshape(4, -1, _EB)
    dstb = dst.reshape(-1, _EB)

    hist = _hist_sc(dstb, _NR).reshape(2, _NR, _HW)
    x = _pad_rows(jnp.concatenate([featA, featB], axis=0), _NR)
    a, hp4 = _prep_tc(hist, x)

    s4 = _scatter_sc(hp4.reshape(4 * _NR, 64), srcg, dstb, _NR)
    h1, hp4b = _layer1_tc(s4.reshape(4, _NR, 64), a, x, W1)
    s4b = _scatter_sc(hp4b.reshape(4 * _NR, 64), srcg, dstb, _NR)
    return _layer2_tc(s4b.reshape(4, _NR, 64), a, h1, x, W2, emit_chunked)


def kernel(ui_edge_index, ub_edge_index, bi_edge_index, users_feature,
           items_feature, bundles_feature, W1_item, W2_item, W1_bundle,
           W2_bundle, gate_W, gate_b):
    out_ui, ui_chunked = _level(ui_edge_index, users_feature, items_feature,
                                W1_item, W2_item, True)
    (out_ub,) = _level(ub_edge_index, users_feature, bundles_feature,
                       W1_bundle, W2_bundle, False)

    b_idx, i_idx = bi_edge_index[0], bi_edge_index[1]
    offs = (jnp.arange(4, dtype=jnp.int32) * _NR)[:, None]
    psrcg = ((i_idx + _U)[None, :] + offs).reshape(4, -1, _EB)
    pdstb = b_idx.reshape(-1, _EB)
    cnt = _hist_sc(pdstb, _NRB).reshape(2, _NRB, _HW)
    pooled4 = _scatter_sc(ui_chunked.reshape(4 * _NR, 64), psrcg, pdstb, _NRB)

    users_il = _pad_rows(out_ui[:_U], _NRB)
    users_bl = _pad_rows(out_ub[:_U], _NRB)
    bundles_bl = _pad_rows(out_ub[_U:_U + _B], _NRB)
    uf = _pad_rows(users_feature, _NRB)
    bf = _pad_rows(bundles_feature, _NRB)

    users_out = _gate_tc(users_il, None, None, users_bl, uf, gate_W, gate_b)
    bundles_out = _gate_tc(None, pooled4.reshape(4, _NRB, 64), cnt,
                           bundles_bl, bf, gate_W, gate_b)
    return jnp.concatenate([users_out[:_U], bundles_out[:_B]], axis=0)
